# baseline (speedup 1.0000x reference)
import jax
import jax.numpy as jnp
from jax.experimental import pallas as pl
from jax.experimental.pallas import tpu as pltpu

_E = 16
_C = 320
_D = 1024
_N = 4096
_BLK = 256
_R = _BLK * _E


def _router_body(x_ref, w_ref, disp_ref, comb_ref, counts_ref):
    blk = x_ref.shape[0]

    @pl.when(pl.program_id(0) == 0)
    def _init():
        counts_ref[...] = jnp.zeros_like(counts_ref)

    logits = jnp.dot(x_ref[...], w_ref[...], preferred_element_type=jnp.float32)
    m = jnp.max(logits, axis=-1, keepdims=True)
    e = jnp.exp(logits - m)
    probs = e / jnp.sum(e, axis=-1, keepdims=True)
    gate = jnp.max(probs, axis=-1, keepdims=True)
    iota_e = jax.lax.broadcasted_iota(jnp.int32, (blk, _E), 1)
    expert = jnp.min(jnp.where(probs == gate, iota_e, _E), axis=-1, keepdims=True)
    mask = (iota_e == expert).astype(jnp.float32)

    r = jax.lax.broadcasted_iota(jnp.int32, (blk, blk), 0)
    c = jax.lax.broadcasted_iota(jnp.int32, (blk, blk), 1)
    tril = (r >= c).astype(jnp.float32)
    csum = jnp.dot(tril, mask, preferred_element_type=jnp.float32)
    pos_all = csum + counts_ref[...]
    counts_ref[...] = counts_ref[...] + csum[blk - 1 : blk, :]
    pos = jnp.sum(pos_all * mask, axis=-1, keepdims=True)
    keep = pos < float(_C)
    pos_m = jnp.where(keep, pos, -1.0)

    rr = jax.lax.broadcasted_iota(jnp.int32, (_R, blk), 0)
    rc = jax.lax.broadcasted_iota(jnp.int32, (_R, blk), 1)
    rep = ((rr >> 4) == rc).astype(jnp.float32)
    hi64 = jnp.floor(pos_m * (1.0 / 64.0)) * 64.0
    lo = pos_m - hi64
    g1 = gate.astype(jnp.bfloat16).astype(jnp.float32)
    g2 = gate - g1
    cols = jnp.concatenate(
        [hi64, lo, expert.astype(jnp.float32), g1, g2], axis=1)
    z = jnp.dot(rep, cols, preferred_element_type=jnp.float32)
    pos_col = z[:, 0:1] + z[:, 1:2]
    exp_col = z[:, 2:3]
    gate_col = z[:, 3:4] + z[:, 4:5]

    e_row = jax.lax.broadcasted_iota(jnp.int32, (_R, 1), 0) & (_E - 1)
    pos_sel = jnp.where(exp_col == e_row.astype(jnp.float32), pos_col, -1.0)

    iota_c = jax.lax.broadcasted_iota(jnp.int32, (1, _C), 1).astype(jnp.float32)
    disp = (iota_c == pos_sel).astype(jnp.float32)
    comb = disp * gate_col
    disp_ref[...] = disp.reshape(blk, _E, _C)
    comb_ref[...] = comb.reshape(blk, _E, _C)


def kernel(inputs, W):
    disp, comb = pl.pallas_call(
        _router_body,
        grid=(_N // _BLK,),
        in_specs=[
            pl.BlockSpec((_BLK, _D), lambda i: (i, 0)),
            pl.BlockSpec((_D, _E), lambda i: (0, 0)),
        ],
        out_specs=[
            pl.BlockSpec((_BLK, _E, _C), lambda i: (i, 0, 0)),
            pl.BlockSpec((_BLK, _E, _C), lambda i: (i, 0, 0)),
        ],
        out_shape=[
            jax.ShapeDtypeStruct((_N, _E, _C), jnp.float32),
            jax.ShapeDtypeStruct((_N, _E, _C), jnp.float32),
        ],
        scratch_shapes=[pltpu.VMEM((1, _E), jnp.float32)],
        compiler_params=pltpu.CompilerParams(
            dimension_semantics=("arbitrary",)
        ),
    )(inputs, W)
    return disp, comb


# speedup vs baseline: 4.0796x; 4.0796x over previous
import jax
import jax.numpy as jnp
from jax.experimental import pallas as pl
from jax.experimental.pallas import tpu as pltpu

_E = 16
_C = 320
_D = 1024
_N = 4096
_B = 512


def _router_body(x_ref, w_ref, disp_ref, comb_ref, counts_ref):
    @pl.when(pl.program_id(0) == 0)
    def _init():
        counts_ref[...] = jnp.zeros_like(counts_ref)

    lg = jax.lax.dot_general(
        w_ref[...], x_ref[...], (((0,), (1,)), ((), ())),
        preferred_element_type=jnp.float32)
    m = jnp.max(lg, axis=0, keepdims=True)
    e = jnp.exp(lg - m)
    probs = e / jnp.sum(e, axis=0, keepdims=True)
    gate = jnp.max(probs, axis=0, keepdims=True)
    iota_e = jax.lax.broadcasted_iota(jnp.int32, (_E, _B), 0)
    expert = jnp.min(jnp.where(probs == gate, iota_e, _E), axis=0,
                     keepdims=True)
    mask = (iota_e == expert).astype(jnp.float32)

    r = jax.lax.broadcasted_iota(jnp.int32, (_B, _B), 0)
    c = jax.lax.broadcasted_iota(jnp.int32, (_B, _B), 1)
    triu = (r <= c).astype(jnp.float32)
    csum = jnp.dot(mask, triu, preferred_element_type=jnp.float32)
    pos = csum + counts_ref[...]
    counts_ref[...] = counts_ref[...] + csum[:, _B - 1 : _B]
    p_sel = jnp.where((mask > 0.0) & (pos < float(_C)), pos, -1.0)

    iota_c = jax.lax.broadcasted_iota(jnp.int32, (_C, 1), 0).astype(jnp.float32)
    for ex in range(_E):
        row = p_sel[ex : ex + 1, :]
        d = (iota_c == row).astype(jnp.float32)
        disp_ref[ex] = d
        comb_ref[ex] = d * gate


def kernel(inputs, W):
    disp_t, comb_t = pl.pallas_call(
        _router_body,
        grid=(_N // _B,),
        in_specs=[
            pl.BlockSpec((_B, _D), lambda i: (i, 0)),
            pl.BlockSpec((_D, _E), lambda i: (0, 0)),
        ],
        out_specs=[
            pl.BlockSpec((_E, _C, _B), lambda i: (0, 0, i)),
            pl.BlockSpec((_E, _C, _B), lambda i: (0, 0, i)),
        ],
        out_shape=[
            jax.ShapeDtypeStruct((_E, _C, _N), jnp.float32),
            jax.ShapeDtypeStruct((_E, _C, _N), jnp.float32),
        ],
        scratch_shapes=[pltpu.VMEM((_E, 1), jnp.float32)],
        compiler_params=pltpu.CompilerParams(
            dimension_semantics=("arbitrary",)
        ),
    )(inputs, W)
    return disp_t.transpose(2, 0, 1), comb_t.transpose(2, 0, 1)
